# baseline (speedup 1.0000x reference)
import functools

import jax
import jax.numpy as jnp
from jax import lax
from jax.experimental import pallas as pl
from jax.experimental.pallas import tpu as pltpu
from jax.experimental.pallas import tpu_sc as plsc

N = 10000
E = 320000
D = 128
N_Q = 4
D_QK = 16
D_V = 16
ATT_DIM = 128
MID = 160

NC = 2
NS = 16
NW = NC * NS
L = 16

NHALF = 10
EH = E // NHALF

TWI = D
EPW = EH // NW
CH = 200
NCHUNK = EPW // CH


def _sc_gather_body(t_hbm, m_hbm, em_hbm, sidx_hbm, ridx_hbm,
                    out_s, out_r, out_m,
                    sbuf, rbuf, embuf, mbuf, mtab, srows, rrows, sem1, sem2):
    wid = lax.axis_index("s") * NC + lax.axis_index("c")
    base = wid * EPW
    pltpu.sync_copy(m_hbm, mtab)

    def chunk_body(c, carry):
        off = base + c * CH
        pltpu.sync_copy(sidx_hbm.at[pl.ds(off, CH)], sbuf)
        pltpu.sync_copy(ridx_hbm.at[pl.ds(off, CH)], rbuf)
        pltpu.sync_copy(em_hbm.at[pl.ds(off, CH)], embuf)
        cp_s = pltpu.async_copy(t_hbm.at[sbuf], srows, sem1)
        cp_r = pltpu.async_copy(t_hbm.at[rbuf], rrows, sem2)
        for v in range(CH // L):
            iv = sbuf[pl.ds(v * L, L)]
            mv = plsc.load_gather(mtab, [iv])
            mbuf[pl.ds(v * L, L)] = mv * embuf[pl.ds(v * L, L)]
        if CH % L:
            iv = sbuf[pl.ds(CH - L, L)]
            mv = plsc.load_gather(mtab, [iv])
            mbuf[pl.ds(CH - L, L)] = mv * embuf[pl.ds(CH - L, L)]
        cp_s.wait()
        cp_r.wait()
        pltpu.sync_copy(srows, out_s.at[pl.ds(off, CH)])
        pltpu.sync_copy(rrows, out_r.at[pl.ds(off, CH)])
        pltpu.sync_copy(mbuf, out_m.at[pl.ds(off, CH)])
        return carry

    lax.fori_loop(0, NCHUNK, chunk_body, 0)


def _sc_gather(t, m_flat, emask, sidx, ridx):
    k = functools.partial(
        pl.kernel,
        out_type=(
            jax.ShapeDtypeStruct((EH, TWI), jnp.int32),
            jax.ShapeDtypeStruct((EH, TWI), jnp.int32),
            jax.ShapeDtypeStruct((EH,), jnp.float32),
        ),
        mesh=plsc.VectorSubcoreMesh(core_axis_name="c", subcore_axis_name="s"),
        compiler_params=pltpu.CompilerParams(needs_layout_passes=False),
        scratch_types=[
            pltpu.VMEM((CH,), jnp.int32),
            pltpu.VMEM((CH,), jnp.int32),
            pltpu.VMEM((CH,), jnp.float32),
            pltpu.VMEM((CH,), jnp.float32),
            pltpu.VMEM((N,), jnp.float32),
            pltpu.VMEM((CH, TWI), jnp.int32),
            pltpu.VMEM((CH, TWI), jnp.int32),
            pltpu.SemaphoreType.DMA,
            pltpu.SemaphoreType.DMA,
        ],
    )(_sc_gather_body)
    return k(t, m_flat, emask, sidx, ridx)


BE = 640
NBLK = EH // BE


def _tc_edge_body(gs_ref, gr_ref, me_ref,
                  wkq_ref, bkq_ref, wv1_ref, bv1_ref,
                  wy2_ref, by2_ref, wn2_ref, bn2_ref,
                  bias64_ref, watt_ref, batt_ref, gsum_ref, gb_ref,
                  attm_ref, dec_ref):
    himask = jnp.int32(-65536)
    gs_i = gs_ref[...]
    gr_i = gr_ref[...]
    ps = lax.bitcast_convert_type(lax.shift_left(gs_i, 16), jnp.float32)
    hs = lax.bitcast_convert_type(gs_i & himask, jnp.float32)
    pr = lax.bitcast_convert_type(lax.shift_left(gr_i, 16), jnp.float32)
    hr = lax.bitcast_convert_type(gr_i & himask, jnp.float32)
    me = jnp.transpose(me_ref[...].reshape(1, BE), (1, 0))
    psr = ps - pr
    hsr = hs - hr
    invp = lax.rsqrt(jnp.maximum(
        jnp.sum(psr * psr, axis=1, keepdims=True), 1e-16))
    invh = lax.rsqrt(jnp.maximum(
        jnp.sum(hsr * hsr, axis=1, keepdims=True), 1e-16))
    pu = psr * invp
    hu = hsr * invh
    x = jnp.concatenate([ps, pr, hs, hr, pu, hu],
                        axis=1).astype(jnp.bfloat16)
    kq = jnp.dot(x, wkq_ref[...], preferred_element_type=jnp.float32) + bkq_ref[...]
    prod = kq[:, 0:N_Q * D_QK] * kq[:, N_Q * D_QK:2 * N_Q * D_QK]
    dsumb = jnp.dot(prod, gb_ref[...], preferred_element_type=jnp.float32)
    dec64 = 1.0 / (1.0 + jnp.exp(-(dsumb + bias64_ref[...])))
    sr_red = jnp.concatenate([psr, hs], axis=1).astype(jnp.bfloat16)
    hmid = jnp.maximum(
        jnp.dot(sr_red, wv1_ref[...], preferred_element_type=jnp.float32) + bv1_ref[...],
        0.0)
    vy = jnp.maximum(
        jnp.dot(hmid[:, 0:MID], wy2_ref[...], preferred_element_type=jnp.float32) + by2_ref[...],
        0.0)
    vn = jnp.maximum(
        jnp.dot(hmid[:, MID:2 * MID], wn2_ref[...], preferred_element_type=jnp.float32) + bn2_ref[...],
        0.0)
    resp = vn + dec64 * (vy - vn)
    att_sr = jnp.maximum(
        jnp.dot(resp, watt_ref[...], preferred_element_type=jnp.float32) + batt_ref[...],
        0.0)
    attm_ref[...] = att_sr * me
    dec_ref[...] = jnp.dot(
        dec64, gsum_ref[...], preferred_element_type=jnp.float32) * (1.0 / 16.0)


def _tc_edge(gs, gr, me3, wkq, bkq, wv1, bv1, wy2, by2, wn2, bn2,
             bias2, watt, batt, gsum, gexp):
    full = lambda shape: pl.BlockSpec(shape, lambda i: (0, 0))
    return pl.pallas_call(
        _tc_edge_body,
        grid=(NBLK,),
        in_specs=[
            pl.BlockSpec((BE, TWI), lambda i: (i, 0)),
            pl.BlockSpec((BE, TWI), lambda i: (i, 0)),
            pl.BlockSpec((1, 1, BE), lambda i: (i, 0, 0)),
            full((6 * D, 2 * N_Q * D_QK)),
            full((1, 2 * N_Q * D_QK)),
            full((2 * D, 2 * MID)),
            full((1, 2 * MID)),
            full((MID, N_Q * D_V)),
            full((1, N_Q * D_V)),
            full((MID, N_Q * D_V)),
            full((1, N_Q * D_V)),
            full((1, N_Q * D_V)),
            full((N_Q * D_V, ATT_DIM)),
            full((1, ATT_DIM)),
            full((N_Q * D_V, N_Q)),
            full((N_Q * D_V, N_Q * D_V)),
        ],
        out_specs=[
            pl.BlockSpec((BE, ATT_DIM), lambda i: (i, 0)),
            pl.BlockSpec((BE, N_Q), lambda i: (i, 0)),
        ],
        out_shape=[
            jax.ShapeDtypeStruct((EH, ATT_DIM), jnp.float32),
            jax.ShapeDtypeStruct((EH, N_Q), jnp.float32),
        ],
    )(gs, gr, me3, wkq, bkq, wv1, bv1, wy2, by2, wn2, bn2,
      bias2, watt, batt, gsum, gexp)


NR = 320
NPAD = NW * NR
ICH = 2000
NICH = EH // ICH
CAP = 8192
DRAIN = 4096
GB = 128
EID_BITS = 19
EID_MASK = (1 << EID_BITS) - 1


def _sc_scatter_body(attm_hbm, ridx_hbm, out_hbm,
                     ribuf, ribuf2, comb_buf, gid_buf, gid_buf2, rows, rows2,
                     acc, gsem0, gsem1, isem0, isem1):
    wid = lax.axis_index("s") * NC + lax.axis_index("c")
    lo = wid * NR
    lane = lax.iota(jnp.int32, L)

    def zero_body(i, carry):
        for j in range(ATT_DIM // L):
            acc[i, pl.ds(j * L, L)] = jnp.zeros((L,), jnp.float32)
        return carry

    lax.fori_loop(0, NR + 1, zero_body, 0)

    padv = jnp.full((L,), NR << EID_BITS, jnp.int32)

    def _issue(b, gid, rowbuf, gsem):
        bbase = b * GB
        for v in range(GB // L):
            cv = comb_buf[pl.ds(bbase + v * L, L)]
            gid[pl.ds(v * L, L)] = cv & EID_MASK
        pltpu.async_copy(attm_hbm.at[gid], rowbuf, gsem)

    def _process(b, rowbuf):
        bbase = b * GB

        def grp_body(g, c2):
            cv = comb_buf[pl.ds(bbase + g * L, L)]
            for li in range(L):
                comb = cv[li]
                dl = lax.shift_right_logical(comb, EID_BITS)
                for j in range(ATT_DIM // L):
                    cur = acc[dl, pl.ds(j * L, L)]
                    acc[dl, pl.ds(j * L, L)] = jnp.maximum(
                        cur, rowbuf[g * L + li, pl.ds(j * L, L)])
            return c2

        lax.fori_loop(0, GB // L, grp_body, 0)

    def drain_all(ptr):
        for k in range(GB // L):
            comb_buf[pl.ds(ptr + k * L, L)] = padv
        nb = (ptr + GB - 1) // GB

        @pl.when(nb > 0)
        def _():
            _issue(0, gid_buf, rows, gsem0)

        def pair_body(k2, carry):
            b = 2 * k2

            @pl.when(b + 1 < nb)
            def _():
                _issue(b + 1, gid_buf2, rows2, gsem1)

            pltpu.make_async_copy(attm_hbm.at[gid_buf], rows, gsem0).wait()
            _process(b, rows)

            @pl.when(b + 2 < nb)
            def _():
                _issue(b + 2, gid_buf, rows, gsem0)

            @pl.when(b + 1 < nb)
            def _():
                pltpu.make_async_copy(attm_hbm.at[gid_buf2], rows2, gsem1).wait()
                _process(b + 1, rows2)

            return carry

        lax.fori_loop(0, (nb + 1) // 2, pair_body, 0)
        return jnp.int32(0)

    U = 5

    def scan_chunk(c, buf, ptr):
        def scan_body(u, pcur):
            entries = []
            for t in range(U):
                v = u * U + t
                idxv = buf[pl.ds(v * L, L)]
                mask = (idxv >= lo) & (idxv < lo + NR)
                eid = c * ICH + v * L + lane
                comb = eid | lax.shift_left(idxv - lo, EID_BITS)
                cnt = plsc.all_reduce_population_count(mask)[0]
                entries.append((mask, comb, cnt))
            for mask, comb, cnt in entries:
                plsc.store_compressed(comb_buf.at[pl.ds(pcur, L)], comb, mask=mask)
                pcur = pcur + cnt
            return pcur

        ptr = lax.fori_loop(0, ICH // L // U, scan_body, ptr)
        return lax.cond(ptr >= DRAIN, drain_all, lambda q: q, ptr)

    pltpu.async_copy(ridx_hbm.at[pl.ds(0, ICH)], ribuf, isem0)

    def pair_body(k, ptr):
        c = 2 * k
        pltpu.async_copy(ridx_hbm.at[pl.ds((c + 1) * ICH, ICH)], ribuf2, isem1)
        pltpu.make_async_copy(ridx_hbm.at[pl.ds(c * ICH, ICH)], ribuf, isem0).wait()
        ptr = scan_chunk(c, ribuf, ptr)

        @pl.when(c + 2 < NICH)
        def _():
            pltpu.async_copy(ridx_hbm.at[pl.ds((c + 2) * ICH, ICH)], ribuf, isem0)

        pltpu.make_async_copy(
            ridx_hbm.at[pl.ds((c + 1) * ICH, ICH)], ribuf2, isem1).wait()
        ptr = scan_chunk(c + 1, ribuf2, ptr)
        return ptr

    ptr = lax.fori_loop(0, NICH // 2, pair_body, jnp.int32(0))
    drain_all(ptr)
    pltpu.sync_copy(acc.at[pl.ds(0, NR)], out_hbm.at[pl.ds(lo, NR)])


def _sc_scatter(attm, ridx):
    k = functools.partial(
        pl.kernel,
        out_type=jax.ShapeDtypeStruct((NPAD, ATT_DIM), jnp.float32),
        mesh=plsc.VectorSubcoreMesh(core_axis_name="c", subcore_axis_name="s"),
        compiler_params=pltpu.CompilerParams(needs_layout_passes=False),
        scratch_types=[
            pltpu.VMEM((ICH,), jnp.int32),
            pltpu.VMEM((ICH,), jnp.int32),
            pltpu.VMEM((CAP,), jnp.int32),
            pltpu.VMEM((GB,), jnp.int32),
            pltpu.VMEM((GB,), jnp.int32),
            pltpu.VMEM((GB, ATT_DIM), jnp.float32),
            pltpu.VMEM((GB, ATT_DIM), jnp.float32),
            pltpu.VMEM((NR + 1, ATT_DIM), jnp.float32),
            pltpu.SemaphoreType.DMA,
            pltpu.SemaphoreType.DMA,
            pltpu.SemaphoreType.DMA,
            pltpu.SemaphoreType.DMA,
        ],
    )(_sc_scatter_body)
    return k(attm, ridx)


BN = 1000
NBLK4 = N // BN


def _tc_final_body(*refs):
    a_refs = refs[:NHALF]
    m_ref, wc_ref, bc_ref, out_ref = refs[NHALF:]
    amax = a_refs[0][...]
    for r in a_refs[1:]:
        amax = jnp.maximum(amax, r[...])
    y = jnp.dot(amax, wc_ref[...], preferred_element_type=jnp.float32) + bc_ref[...]
    out_ref[...] = jnp.maximum(y, 0.0) * m_ref[...]


def _tc_final(att_parts, m, wc, bc2):
    return pl.pallas_call(
        _tc_final_body,
        grid=(NBLK4,),
        in_specs=(
            [pl.BlockSpec((BN, ATT_DIM), lambda i: (i, 0))] * NHALF
            + [
                pl.BlockSpec((BN, 1), lambda i: (i, 0)),
                pl.BlockSpec((D, D), lambda i: (0, 0)),
                pl.BlockSpec((1, D), lambda i: (0, 0)),
            ]
        ),
        out_specs=pl.BlockSpec((BN, D), lambda i: (i, 0)),
        out_shape=jax.ShapeDtypeStruct((N, D), jnp.float32),
    )(*att_parts, m, wc, bc2)


def _restruct_kq(w):
    b = [w[i * D:(i + 1) * D] for i in range(8)]
    return jnp.concatenate(
        [b[0] + b[2], b[1] - b[2], b[4] + b[6], b[5] - b[6], b[3], b[7]], axis=0)


def kernel(h, m, edge_index, edge_mask, p, W_k, b_k, W_q, b_q,
           W_vy1, b_vy1, W_vy2, b_vy2, W_vn1, b_vn1, W_vn2, b_vn2,
           bias, W_att, b_att, W_c, b_c):
    sidx = edge_index[0].astype(jnp.int32)
    ridx = edge_index[1].astype(jnp.int32)
    pbits = lax.bitcast_convert_type(
        p.astype(jnp.bfloat16), jnp.uint16).astype(jnp.uint32)
    hbits = lax.bitcast_convert_type(
        h.astype(jnp.bfloat16), jnp.uint16).astype(jnp.uint32)
    ti = lax.bitcast_convert_type(pbits | (hbits << 16), jnp.int32)
    m_flat = m[:, 0]

    wkq = jnp.concatenate(
        [_restruct_kq(W_k), _restruct_kq(W_q)], axis=1).astype(jnp.bfloat16)
    bkq = jnp.concatenate([b_k, b_q])[None, :]
    wv1 = jnp.concatenate([W_vy1, W_vn1], axis=1).astype(jnp.bfloat16)
    bv1 = jnp.concatenate([b_vy1, b_vn1])[None, :]
    head = jnp.arange(N_Q * D_V, dtype=jnp.int32) // D_V
    gsum = (head[:, None] == jnp.arange(N_Q)[None, :]).astype(jnp.float32)
    gb = gsum @ gsum.T
    bias64 = (bias[None, :] @ gsum.T)

    att_halves = []
    dec_halves = []
    for half in range(NHALF):
        sl = slice(half * EH, (half + 1) * EH)
        gs, gr, me = _sc_gather(ti, m_flat, edge_mask[sl], sidx[sl], ridx[sl])
        attm, dec = _tc_edge(gs, gr, me.reshape(NBLK, 1, BE),
                             wkq, bkq, wv1, bv1,
                             W_vy2, b_vy2[None, :], W_vn2, b_vn2[None, :],
                             bias64, W_att, b_att[None, :], gsum, gb)
        att_halves.append(_sc_scatter(attm, ridx[sl]))
        dec_halves.append(dec)

    att = _tc_final([a[:N] for a in att_halves], m, W_c, b_c[None, :])
    dec = jnp.concatenate(dec_halves, axis=0)
    return (att, dec)


# speedup vs baseline: 1.3788x; 1.3788x over previous
import functools

import jax
import jax.numpy as jnp
from jax import lax
from jax.experimental import pallas as pl
from jax.experimental.pallas import tpu as pltpu
from jax.experimental.pallas import tpu_sc as plsc

N = 10000
E = 320000
D = 128
N_Q = 4
D_QK = 16
D_V = 16
ATT_DIM = 128
MID = 160

NC = 2
NS = 16
NW = NC * NS
L = 16

NHALF = 5
EH = E // NHALF

TWI = D
EPW = EH // NW
CH = 200
NCHUNK = EPW // CH


def _sc_gather_body(t_hbm, m_hbm, em_hbm, sidx_hbm, ridx_hbm,
                    out_s, out_r, out_m,
                    sbuf, rbuf, embuf, mbuf, mtab, srows, rrows, sem1, sem2):
    wid = lax.axis_index("s") * NC + lax.axis_index("c")
    base = wid * EPW
    pltpu.sync_copy(m_hbm, mtab)

    def chunk_body(c, carry):
        off = base + c * CH
        pltpu.sync_copy(sidx_hbm.at[pl.ds(off, CH)], sbuf)
        pltpu.sync_copy(ridx_hbm.at[pl.ds(off, CH)], rbuf)
        pltpu.sync_copy(em_hbm.at[pl.ds(off, CH)], embuf)
        cp_s = pltpu.async_copy(t_hbm.at[sbuf], srows, sem1)
        cp_r = pltpu.async_copy(t_hbm.at[rbuf], rrows, sem2)
        for v in range(CH // L):
            iv = sbuf[pl.ds(v * L, L)]
            mv = plsc.load_gather(mtab, [iv])
            mbuf[pl.ds(v * L, L)] = mv * embuf[pl.ds(v * L, L)]
        if CH % L:
            iv = sbuf[pl.ds(CH - L, L)]
            mv = plsc.load_gather(mtab, [iv])
            mbuf[pl.ds(CH - L, L)] = mv * embuf[pl.ds(CH - L, L)]
        cp_s.wait()
        cp_r.wait()
        pltpu.sync_copy(srows, out_s.at[pl.ds(off, CH)])
        pltpu.sync_copy(rrows, out_r.at[pl.ds(off, CH)])
        pltpu.sync_copy(mbuf, out_m.at[pl.ds(off, CH)])
        return carry

    lax.fori_loop(0, NCHUNK, chunk_body, 0)


def _sc_gather(t, m_flat, emask, sidx, ridx):
    k = functools.partial(
        pl.kernel,
        out_type=(
            jax.ShapeDtypeStruct((EH, TWI), jnp.int32),
            jax.ShapeDtypeStruct((EH, TWI), jnp.int32),
            jax.ShapeDtypeStruct((EH,), jnp.float32),
        ),
        mesh=plsc.VectorSubcoreMesh(core_axis_name="c", subcore_axis_name="s"),
        compiler_params=pltpu.CompilerParams(needs_layout_passes=False),
        scratch_types=[
            pltpu.VMEM((CH,), jnp.int32),
            pltpu.VMEM((CH,), jnp.int32),
            pltpu.VMEM((CH,), jnp.float32),
            pltpu.VMEM((CH,), jnp.float32),
            pltpu.VMEM((N,), jnp.float32),
            pltpu.VMEM((CH, TWI), jnp.int32),
            pltpu.VMEM((CH, TWI), jnp.int32),
            pltpu.SemaphoreType.DMA,
            pltpu.SemaphoreType.DMA,
        ],
    )(_sc_gather_body)
    return k(t, m_flat, emask, sidx, ridx)


BE = 640
NBLK = EH // BE


def _tc_edge_body(gs_ref, gr_ref, me_ref,
                  wkq_ref, bkq_ref, wv1_ref, bv1_ref,
                  wy2_ref, by2_ref, wn2_ref, bn2_ref,
                  bias64_ref, watt_ref, batt_ref, gsum_ref, gb_ref,
                  attm_ref, dec_ref):
    himask = jnp.int32(-65536)
    gs_i = gs_ref[...]
    gr_i = gr_ref[...]
    ps = lax.bitcast_convert_type(lax.shift_left(gs_i, 16), jnp.float32)
    hs = lax.bitcast_convert_type(gs_i & himask, jnp.float32)
    pr = lax.bitcast_convert_type(lax.shift_left(gr_i, 16), jnp.float32)
    hr = lax.bitcast_convert_type(gr_i & himask, jnp.float32)
    me = jnp.transpose(me_ref[...].reshape(1, BE), (1, 0))
    psr = ps - pr
    hsr = hs - hr
    invp = lax.rsqrt(jnp.maximum(
        jnp.sum(psr * psr, axis=1, keepdims=True), 1e-16))
    invh = lax.rsqrt(jnp.maximum(
        jnp.sum(hsr * hsr, axis=1, keepdims=True), 1e-16))
    pu = psr * invp
    hu = hsr * invh
    x = jnp.concatenate([ps, pr, hs, hr, pu, hu],
                        axis=1).astype(jnp.bfloat16)
    kq = jnp.dot(x, wkq_ref[...], preferred_element_type=jnp.float32) + bkq_ref[...]
    prod = kq[:, 0:N_Q * D_QK] * kq[:, N_Q * D_QK:2 * N_Q * D_QK]
    dsumb = jnp.dot(prod, gb_ref[...], preferred_element_type=jnp.float32)
    dec64 = 1.0 / (1.0 + jnp.exp(-(dsumb + bias64_ref[...])))
    sr_red = jnp.concatenate([psr, hs], axis=1).astype(jnp.bfloat16)
    hmid = jnp.maximum(
        jnp.dot(sr_red, wv1_ref[...], preferred_element_type=jnp.float32) + bv1_ref[...],
        0.0)
    vy = jnp.maximum(
        jnp.dot(hmid[:, 0:MID], wy2_ref[...], preferred_element_type=jnp.float32) + by2_ref[...],
        0.0)
    vn = jnp.maximum(
        jnp.dot(hmid[:, MID:2 * MID], wn2_ref[...], preferred_element_type=jnp.float32) + bn2_ref[...],
        0.0)
    resp = vn + dec64 * (vy - vn)
    att_sr = jnp.maximum(
        jnp.dot(resp, watt_ref[...], preferred_element_type=jnp.float32) + batt_ref[...],
        0.0)
    attm_ref[...] = att_sr * me
    dec_ref[...] = jnp.dot(
        dec64, gsum_ref[...], preferred_element_type=jnp.float32) * (1.0 / 16.0)


def _tc_edge(gs, gr, me3, wkq, bkq, wv1, bv1, wy2, by2, wn2, bn2,
             bias2, watt, batt, gsum, gexp):
    full = lambda shape: pl.BlockSpec(shape, lambda i: (0, 0))
    return pl.pallas_call(
        _tc_edge_body,
        grid=(NBLK,),
        in_specs=[
            pl.BlockSpec((BE, TWI), lambda i: (i, 0)),
            pl.BlockSpec((BE, TWI), lambda i: (i, 0)),
            pl.BlockSpec((1, 1, BE), lambda i: (i, 0, 0)),
            full((6 * D, 2 * N_Q * D_QK)),
            full((1, 2 * N_Q * D_QK)),
            full((2 * D, 2 * MID)),
            full((1, 2 * MID)),
            full((MID, N_Q * D_V)),
            full((1, N_Q * D_V)),
            full((MID, N_Q * D_V)),
            full((1, N_Q * D_V)),
            full((1, N_Q * D_V)),
            full((N_Q * D_V, ATT_DIM)),
            full((1, ATT_DIM)),
            full((N_Q * D_V, N_Q)),
            full((N_Q * D_V, N_Q * D_V)),
        ],
        out_specs=[
            pl.BlockSpec((BE, ATT_DIM), lambda i: (i, 0)),
            pl.BlockSpec((BE, N_Q), lambda i: (i, 0)),
        ],
        out_shape=[
            jax.ShapeDtypeStruct((EH, ATT_DIM), jnp.float32),
            jax.ShapeDtypeStruct((EH, N_Q), jnp.float32),
        ],
    )(gs, gr, me3, wkq, bkq, wv1, bv1, wy2, by2, wn2, bn2,
      bias2, watt, batt, gsum, gexp)


NR = 320
NPAD = NW * NR
ICH = 2000
NICH = EH // ICH
CAP = 8192
DRAIN = 4096
GB = 128
EID_BITS = 19
EID_MASK = (1 << EID_BITS) - 1


def _sc_scatter_body(attm_hbm, ridx_hbm, out_hbm,
                     ribuf, ribuf2, comb_buf, gid_buf, gid_buf2, rows, rows2,
                     acc, gsem0, gsem1, isem0, isem1):
    wid = lax.axis_index("s") * NC + lax.axis_index("c")
    lo = wid * NR
    lane = lax.iota(jnp.int32, L)

    def zero_body(i, carry):
        for j in range(ATT_DIM // L):
            acc[i, pl.ds(j * L, L)] = jnp.zeros((L,), jnp.float32)
        return carry

    lax.fori_loop(0, NR + 1, zero_body, 0)

    padv = jnp.full((L,), NR << EID_BITS, jnp.int32)

    def _issue(b, gid, rowbuf, gsem):
        bbase = b * GB
        for v in range(GB // L):
            cv = comb_buf[pl.ds(bbase + v * L, L)]
            gid[pl.ds(v * L, L)] = cv & EID_MASK
        pltpu.async_copy(attm_hbm.at[gid], rowbuf, gsem)

    def _process(b, rowbuf):
        bbase = b * GB

        def grp_body(g, c2):
            cv = comb_buf[pl.ds(bbase + g * L, L)]
            for li in range(L):
                comb = cv[li]
                dl = lax.shift_right_logical(comb, EID_BITS)
                for j in range(ATT_DIM // L):
                    cur = acc[dl, pl.ds(j * L, L)]
                    acc[dl, pl.ds(j * L, L)] = jnp.maximum(
                        cur, rowbuf[g * L + li, pl.ds(j * L, L)])
            return c2

        lax.fori_loop(0, GB // L, grp_body, 0)

    def drain_all(ptr):
        for k in range(GB // L):
            comb_buf[pl.ds(ptr + k * L, L)] = padv
        nb = (ptr + GB - 1) // GB

        @pl.when(nb > 0)
        def _():
            _issue(0, gid_buf, rows, gsem0)

        def pair_body(k2, carry):
            b = 2 * k2

            @pl.when(b + 1 < nb)
            def _():
                _issue(b + 1, gid_buf2, rows2, gsem1)

            pltpu.make_async_copy(attm_hbm.at[gid_buf], rows, gsem0).wait()
            _process(b, rows)

            @pl.when(b + 2 < nb)
            def _():
                _issue(b + 2, gid_buf, rows, gsem0)

            @pl.when(b + 1 < nb)
            def _():
                pltpu.make_async_copy(attm_hbm.at[gid_buf2], rows2, gsem1).wait()
                _process(b + 1, rows2)

            return carry

        lax.fori_loop(0, (nb + 1) // 2, pair_body, 0)
        return jnp.int32(0)

    U = 5

    def scan_chunk(c, buf, ptr):
        def scan_body(u, pcur):
            entries = []
            for t in range(U):
                v = u * U + t
                idxv = buf[pl.ds(v * L, L)]
                mask = (idxv >= lo) & (idxv < lo + NR)
                eid = c * ICH + v * L + lane
                comb = eid | lax.shift_left(idxv - lo, EID_BITS)
                cnt = plsc.all_reduce_population_count(mask)[0]
                entries.append((mask, comb, cnt))
            for mask, comb, cnt in entries:
                plsc.store_compressed(comb_buf.at[pl.ds(pcur, L)], comb, mask=mask)
                pcur = pcur + cnt
            return pcur

        ptr = lax.fori_loop(0, ICH // L // U, scan_body, ptr)
        return lax.cond(ptr >= DRAIN, drain_all, lambda q: q, ptr)

    pltpu.async_copy(ridx_hbm.at[pl.ds(0, ICH)], ribuf, isem0)

    def pair_body(k, ptr):
        c = 2 * k
        pltpu.async_copy(ridx_hbm.at[pl.ds((c + 1) * ICH, ICH)], ribuf2, isem1)
        pltpu.make_async_copy(ridx_hbm.at[pl.ds(c * ICH, ICH)], ribuf, isem0).wait()
        ptr = scan_chunk(c, ribuf, ptr)

        @pl.when(c + 2 < NICH)
        def _():
            pltpu.async_copy(ridx_hbm.at[pl.ds((c + 2) * ICH, ICH)], ribuf, isem0)

        pltpu.make_async_copy(
            ridx_hbm.at[pl.ds((c + 1) * ICH, ICH)], ribuf2, isem1).wait()
        ptr = scan_chunk(c + 1, ribuf2, ptr)
        return ptr

    ptr = lax.fori_loop(0, NICH // 2, pair_body, jnp.int32(0))
    drain_all(ptr)
    pltpu.sync_copy(acc.at[pl.ds(0, NR)], out_hbm.at[pl.ds(lo, NR)])


def _sc_scatter(attm, ridx):
    k = functools.partial(
        pl.kernel,
        out_type=jax.ShapeDtypeStruct((NPAD, ATT_DIM), jnp.float32),
        mesh=plsc.VectorSubcoreMesh(core_axis_name="c", subcore_axis_name="s"),
        compiler_params=pltpu.CompilerParams(needs_layout_passes=False),
        scratch_types=[
            pltpu.VMEM((ICH,), jnp.int32),
            pltpu.VMEM((ICH,), jnp.int32),
            pltpu.VMEM((CAP,), jnp.int32),
            pltpu.VMEM((GB,), jnp.int32),
            pltpu.VMEM((GB,), jnp.int32),
            pltpu.VMEM((GB, ATT_DIM), jnp.float32),
            pltpu.VMEM((GB, ATT_DIM), jnp.float32),
            pltpu.VMEM((NR + 1, ATT_DIM), jnp.float32),
            pltpu.SemaphoreType.DMA,
            pltpu.SemaphoreType.DMA,
            pltpu.SemaphoreType.DMA,
            pltpu.SemaphoreType.DMA,
        ],
    )(_sc_scatter_body)
    return k(attm, ridx)


BN = 1000
NBLK4 = N // BN


def _tc_final_body(*refs):
    a_refs = refs[:NHALF]
    m_ref, wc_ref, bc_ref, out_ref = refs[NHALF:]
    amax = a_refs[0][...]
    for r in a_refs[1:]:
        amax = jnp.maximum(amax, r[...])
    y = jnp.dot(amax, wc_ref[...], preferred_element_type=jnp.float32) + bc_ref[...]
    out_ref[...] = jnp.maximum(y, 0.0) * m_ref[...]


def _tc_final(att_parts, m, wc, bc2):
    return pl.pallas_call(
        _tc_final_body,
        grid=(NBLK4,),
        in_specs=(
            [pl.BlockSpec((BN, ATT_DIM), lambda i: (i, 0))] * NHALF
            + [
                pl.BlockSpec((BN, 1), lambda i: (i, 0)),
                pl.BlockSpec((D, D), lambda i: (0, 0)),
                pl.BlockSpec((1, D), lambda i: (0, 0)),
            ]
        ),
        out_specs=pl.BlockSpec((BN, D), lambda i: (i, 0)),
        out_shape=jax.ShapeDtypeStruct((N, D), jnp.float32),
    )(*att_parts, m, wc, bc2)


def _restruct_kq(w):
    b = [w[i * D:(i + 1) * D] for i in range(8)]
    return jnp.concatenate(
        [b[0] + b[2], b[1] - b[2], b[4] + b[6], b[5] - b[6], b[3], b[7]], axis=0)


def kernel(h, m, edge_index, edge_mask, p, W_k, b_k, W_q, b_q,
           W_vy1, b_vy1, W_vy2, b_vy2, W_vn1, b_vn1, W_vn2, b_vn2,
           bias, W_att, b_att, W_c, b_c):
    sidx = edge_index[0].astype(jnp.int32)
    ridx = edge_index[1].astype(jnp.int32)
    pbits = lax.bitcast_convert_type(
        p.astype(jnp.bfloat16), jnp.uint16).astype(jnp.uint32)
    hbits = lax.bitcast_convert_type(
        h.astype(jnp.bfloat16), jnp.uint16).astype(jnp.uint32)
    ti = lax.bitcast_convert_type(pbits | (hbits << 16), jnp.int32)
    m_flat = m[:, 0]

    wkq = jnp.concatenate(
        [_restruct_kq(W_k), _restruct_kq(W_q)], axis=1).astype(jnp.bfloat16)
    bkq = jnp.concatenate([b_k, b_q])[None, :]
    wv1 = jnp.concatenate([W_vy1, W_vn1], axis=1).astype(jnp.bfloat16)
    bv1 = jnp.concatenate([b_vy1, b_vn1])[None, :]
    head = jnp.arange(N_Q * D_V, dtype=jnp.int32) // D_V
    gsum = (head[:, None] == jnp.arange(N_Q)[None, :]).astype(jnp.float32)
    gb = gsum @ gsum.T
    bias64 = (bias[None, :] @ gsum.T)

    att_halves = []
    dec_halves = []
    for half in range(NHALF):
        sl = slice(half * EH, (half + 1) * EH)
        gs, gr, me = _sc_gather(ti, m_flat, edge_mask[sl], sidx[sl], ridx[sl])
        attm, dec = _tc_edge(gs, gr, me.reshape(NBLK, 1, BE),
                             wkq, bkq, wv1, bv1,
                             W_vy2, b_vy2[None, :], W_vn2, b_vn2[None, :],
                             bias64, W_att, b_att[None, :], gsum, gb)
        att_halves.append(_sc_scatter(attm, ridx[sl]))
        dec_halves.append(dec)

    att = _tc_final([a[:N] for a in att_halves], m, W_c, b_c[None, :])
    dec = jnp.concatenate(dec_halves, axis=0)
    return (att, dec)
